# baseline (speedup 1.0000x reference)
import functools

import jax
import jax.numpy as jnp
from jax.experimental import pallas as pl
from jax.experimental.pallas import tpu as pltpu


def _chamfer_batch_kernel(n_i, bi, a_ref, b_ref, min_a_ref, min_b_ref,
                          a_scr, b_scr):
    bm = b_ref[0]
    m = bm.shape[0]
    b2 = jnp.sum(bm * bm, axis=1, keepdims=True)
    ones_b = jnp.ones((m, 1), jnp.float32)
    b_scr[:] = jnp.concatenate([-2.0 * bm, ones_b, b2], axis=1)

    a = a_ref[0]
    n = a.shape[0]
    a2 = jnp.sum(a * a, axis=1, keepdims=True)
    ones_a = jnp.ones((n, 1), jnp.float32)
    a_scr[:] = jnp.concatenate([a, a2, ones_a], axis=1)

    def step(i, colmin):
        a_aug = a_scr[pl.ds(i * bi, bi), :]
        d2 = jax.lax.dot_general(
            a_aug, b_scr[:], (((1,), (1,)), ((), ())),
            preferred_element_type=jnp.float32,
        )
        rowmin = jnp.min(d2, axis=1, keepdims=True)
        min_a_ref[0, pl.ds(i * bi, bi), :] = jnp.sqrt(jnp.maximum(rowmin, 0.0))
        return jnp.minimum(colmin, jnp.min(d2, axis=0))

    init = jnp.full((m,), jnp.inf, jnp.float32)
    colmin = jax.lax.fori_loop(0, n_i, step, init)
    min_b_ref[0, 0, :] = jnp.sqrt(jnp.maximum(colmin, 0.0))


def kernel(A, B):
    bt, n, d = A.shape
    m = B.shape[1]
    bi = 512
    n_i = n // bi
    da = d + 2

    min_a, min_b = pl.pallas_call(
        functools.partial(_chamfer_batch_kernel, n_i, bi),
        grid=(bt,),
        in_specs=[
            pl.BlockSpec((1, n, d), lambda b: (b, 0, 0)),
            pl.BlockSpec((1, m, d), lambda b: (b, 0, 0)),
        ],
        out_specs=[
            pl.BlockSpec((1, n, 1), lambda b: (b, 0, 0)),
            pl.BlockSpec((1, 1, m), lambda b: (b, 0, 0)),
        ],
        out_shape=[
            jax.ShapeDtypeStruct((bt, n, 1), jnp.float32),
            jax.ShapeDtypeStruct((bt, 1, m), jnp.float32),
        ],
        scratch_shapes=[
            pltpu.VMEM((n, da), jnp.float32),
            pltpu.VMEM((m, da), jnp.float32),
        ],
    )(A, B)
    min_a = min_a.reshape(bt, n)
    min_b = min_b.reshape(bt, m)
    chamfer = jnp.mean(min_a, axis=1) + jnp.mean(min_b, axis=1)
    return jnp.mean(chamfer) / 12.8


# speedup vs baseline: 1.0025x; 1.0025x over previous
import functools

import jax
import jax.numpy as jnp
from jax.experimental import pallas as pl
from jax.experimental.pallas import tpu as pltpu


def _chamfer_batch_kernel(n_i, bi, a_ref, b_ref, min_a_ref, min_b_ref,
                          a_scr, b_scr):
    bm = b_ref[0]
    m = bm.shape[0]
    b2 = jnp.sum(bm * bm, axis=1, keepdims=True)
    ones_b = jnp.ones((m, 1), jnp.float32)
    b_scr[:] = jnp.concatenate([-2.0 * bm, ones_b, b2], axis=1)

    a = a_ref[0]
    n = a.shape[0]
    a2 = jnp.sum(a * a, axis=1, keepdims=True)
    ones_a = jnp.ones((n, 1), jnp.float32)
    a_scr[:] = jnp.concatenate([a, a2, ones_a], axis=1)

    def step(i, colmin):
        a_aug = a_scr[pl.ds(i * bi, bi), :]
        d2 = jax.lax.dot_general(
            a_aug, b_scr[:], (((1,), (1,)), ((), ())),
            preferred_element_type=jnp.float32,
        )
        rowmin = jnp.min(d2, axis=1, keepdims=True)
        min_a_ref[0, pl.ds(i * bi, bi), :] = jnp.sqrt(jnp.maximum(rowmin, 0.0))
        return jnp.minimum(colmin, jnp.min(d2, axis=0))

    init = jnp.full((m,), jnp.inf, jnp.float32)
    colmin = jax.lax.fori_loop(0, n_i, step, init)
    min_b_ref[0, 0, :] = jnp.sqrt(jnp.maximum(colmin, 0.0))


def kernel(A, B):
    bt, n, d = A.shape
    m = B.shape[1]
    bi = 512
    n_i = n // bi
    da = d + 2

    min_a, min_b = pl.pallas_call(
        functools.partial(_chamfer_batch_kernel, n_i, bi),
        grid=(bt,),
        in_specs=[
            pl.BlockSpec((1, n, d), lambda b: (b, 0, 0)),
            pl.BlockSpec((1, m, d), lambda b: (b, 0, 0)),
        ],
        out_specs=[
            pl.BlockSpec((1, n, 1), lambda b: (b, 0, 0)),
            pl.BlockSpec((1, 1, m), lambda b: (b, 0, 0)),
        ],
        out_shape=[
            jax.ShapeDtypeStruct((bt, n, 1), jnp.float32),
            jax.ShapeDtypeStruct((bt, 1, m), jnp.float32),
        ],
        scratch_shapes=[
            pltpu.VMEM((n, da), jnp.float32),
            pltpu.VMEM((m, da), jnp.float32),
        ],
        compiler_params=pltpu.CompilerParams(
            dimension_semantics=("parallel",),
        ),
    )(A, B)
    min_a = min_a.reshape(bt, n)
    min_b = min_b.reshape(bt, m)
    chamfer = jnp.mean(min_a, axis=1) + jnp.mean(min_b, axis=1)
    return jnp.mean(chamfer) / 12.8
